# Initial kernel scaffold; baseline (speedup 1.0000x reference)
import functools

import jax
import jax.numpy as jnp
from jax import lax
from jax.experimental import pallas as pl
from jax.experimental.pallas import tpu as pltpu
from jax.experimental.pallas import tpu_sc as plsc

N_RBF = 20
R_CUT = 5.0


def _mlp_body(s_ref, w1_ref, b1_ref, w2_ref, b2_ref, o_ref):
    h = jnp.dot(s_ref[...], w1_ref[...], preferred_element_type=jnp.float32)
    h = h + b1_ref[...]
    h = h * jax.nn.sigmoid(h)
    o = jnp.dot(h, w2_ref[...], preferred_element_type=jnp.float32)
    o_ref[...] = o + b2_ref[...]


def _node_mlp(s, W1, b1, W2, b2, bn):
    n, emb = s.shape
    out3 = W2.shape[1]
    grid = n // bn
    return pl.pallas_call(
        _mlp_body,
        grid=(grid,),
        in_specs=[
            pl.BlockSpec((bn, emb), lambda i: (i, 0)),
            pl.BlockSpec((emb, emb), lambda i: (0, 0)),
            pl.BlockSpec((1, emb), lambda i: (0, 0)),
            pl.BlockSpec((emb, out3), lambda i: (0, 0)),
            pl.BlockSpec((1, out3), lambda i: (0, 0)),
        ],
        out_specs=pl.BlockSpec((bn, out3), lambda i: (i, 0)),
        out_shape=jax.ShapeDtypeStruct((n, out3), jnp.float32),
    )(s, W1, b1.reshape(1, emb), W2, b2.reshape(1, out3))


def _make_gather(E, D, n_workers, ch):
    ew = E // n_workers
    nch = ew // ch
    mesh = plsc.VectorSubcoreMesh(core_axis_name="c", subcore_axis_name="s")

    @functools.partial(
        pl.kernel,
        mesh=mesh,
        out_type=[
            jax.ShapeDtypeStruct((E, D), jnp.float32),
            jax.ShapeDtypeStruct((E, D), jnp.float32),
        ],
        scratch_types=[
            pltpu.VMEM((ch,), jnp.int32),
            pltpu.VMEM((ch, D), jnp.float32),
            pltpu.VMEM((ch, D), jnp.float32),
            pltpu.SemaphoreType.DMA,
            pltpu.SemaphoreType.DMA,
        ],
    )
    def gather_k(sp_hbm, v2_hbm, src_hbm, g_hbm, vg_hbm,
                 idx_v, gbuf, vbuf, sem_a, sem_b):
        wid = lax.axis_index("s") * 2 + lax.axis_index("c")
        base = wid * ew

        def body(j, carry):
            e0 = base + j * ch
            pltpu.sync_copy(src_hbm.at[pl.ds(e0, ch)], idx_v)
            cp_a = pltpu.async_copy(sp_hbm.at[idx_v], gbuf, sem_a)
            cp_b = pltpu.async_copy(v2_hbm.at[idx_v], vbuf, sem_b)
            cp_a.wait()
            cp_b.wait()
            pltpu.sync_copy(gbuf, g_hbm.at[pl.ds(e0, ch)])
            pltpu.sync_copy(vbuf, vg_hbm.at[pl.ds(e0, ch)])
            return carry

        lax.fori_loop(0, nch, body, 0)

    return gather_k


def _edge_body(r_ref, rnt_ref, g_ref, vg_ref, wr_ref, br_ref, o_ref):
    be = r_ref.shape[0]
    emb = o_ref.shape[2]
    r = r_ref[...].reshape(be, 1)
    n = lax.broadcasted_iota(jnp.float32, (be, N_RBF), 1) + 1.0
    rbf = jnp.sin(n * (jnp.pi / R_CUT) * r) / r
    rbf_pass = jnp.dot(rbf, wr_ref[...],
                       preferred_element_type=jnp.float32) + br_ref[...]
    f_cut = jnp.where(r <= R_CUT,
                      0.5 * (jnp.cos(jnp.pi * r / R_CUT) + 1.0), 0.0)
    pass_out = rbf_pass * f_cut * g_ref[...]
    coef = pass_out[:, 0:emb]
    ds = pass_out[:, emb:2 * emb]
    rep = pass_out[:, 2 * emb:3 * emb]
    o_ref[0] = ds
    for d in range(3):
        rn_d = rnt_ref[d, :].reshape(be, 1)
        o_ref[d + 1] = vg_ref[:, d * emb:(d + 1) * emb] * coef + rn_d * rep


def _edge_math(r_flat, rn_t, g, vg, Wr, br, be):
    E = r_flat.shape[0]
    emb = g.shape[1] // 3
    grid = E // be
    return pl.pallas_call(
        _edge_body,
        grid=(grid,),
        in_specs=[
            pl.BlockSpec((be,), lambda i: (i,)),
            pl.BlockSpec((3, be), lambda i: (0, i)),
            pl.BlockSpec((be, 3 * emb), lambda i: (i, 0)),
            pl.BlockSpec((be, 3 * emb), lambda i: (i, 0)),
            pl.BlockSpec((N_RBF, 3 * emb), lambda i: (0, 0)),
            pl.BlockSpec((1, 3 * emb), lambda i: (0, 0)),
        ],
        out_specs=pl.BlockSpec((4, be, emb), lambda i: (0, i, 0)),
        out_shape=jax.ShapeDtypeStruct((4, E, emb), jnp.float32),
    )(r_flat, rn_t, g, vg, Wr, br.reshape(1, 3 * emb))


def _make_scatter(E, N, emb, n_workers, ch):
    ew = E // n_workers
    nch = ew // ch
    n_sub = 16
    rows_t = N // n_sub
    zch = 125
    nz = rows_t // zch
    mesh = plsc.VectorSubcoreMesh(core_axis_name="c", subcore_axis_name="s")

    @functools.partial(
        pl.kernel,
        mesh=mesh,
        out_type=jax.ShapeDtypeStruct((2, 4, N, emb), jnp.float32),
        scratch_types=[
            pltpu.VMEM((nch, ch), jnp.int32),
            pltpu.VMEM((ch, emb), jnp.float32),
            pltpu.VMEM((zch, emb), jnp.float32),
            pltpu.VMEM((zch, emb), jnp.float32),
            pltpu.VMEM_SHARED((N, emb), jnp.float32),
        ],
    )
    def scatter_k(d4_hbm, dst3_hbm, part_hbm,
                  idxmat, dbuf, zbuf, obuf, accum):
        cid = lax.axis_index("c")
        sid = lax.axis_index("s")
        wid = sid * 2 + cid
        base = wid * ew

        zero = jnp.zeros((16,), jnp.float32)
        def zrow(i, carry):
            for q in range(emb // 16):
                zbuf[i, pl.ds(q * 16, 16)] = zero
            return carry
        lax.fori_loop(0, zch, zrow, 0)

        pltpu.sync_copy(dst3_hbm.at[wid], idxmat)

        for p in range(4):
            for q in range(nz):
                pltpu.sync_copy(
                    zbuf, accum.at[pl.ds(sid * rows_t + q * zch, zch)])
            plsc.subcore_barrier()

            def body(j, carry):
                e0 = base + j * ch
                pltpu.sync_copy(d4_hbm.at[p, pl.ds(e0, ch)], dbuf)
                pltpu.sync_copy(dbuf, accum.at[idxmat.at[j]], add=True)
                return carry
            lax.fori_loop(0, nch, body, 0)
            plsc.subcore_barrier()

            for q in range(nz):
                r0 = sid * rows_t + q * zch
                pltpu.sync_copy(accum.at[pl.ds(r0, zch)], obuf)
                pltpu.sync_copy(obuf, part_hbm.at[cid, p, pl.ds(r0, zch)])
            plsc.subcore_barrier()

    return scatter_k


def _combine_body(s_ref, v2_ref, part_ref, so_ref, vo_ref):
    so_ref[...] = s_ref[...] + part_ref[0, 0] + part_ref[1, 0]
    dv = [part_ref[0, 1 + d] + part_ref[1, 1 + d] for d in range(3)]
    vo_ref[...] = v2_ref[...] + jnp.concatenate(dv, axis=1)


def _combine(s, v2, part, bn):
    n, emb = s.shape
    grid = n // bn
    return pl.pallas_call(
        _combine_body,
        grid=(grid,),
        in_specs=[
            pl.BlockSpec((bn, emb), lambda i: (i, 0)),
            pl.BlockSpec((bn, 3 * emb), lambda i: (i, 0)),
            pl.BlockSpec((2, 4, bn, emb), lambda i: (0, 0, i, 0)),
        ],
        out_specs=[
            pl.BlockSpec((bn, emb), lambda i: (i, 0)),
            pl.BlockSpec((bn, 3 * emb), lambda i: (i, 0)),
        ],
        out_shape=[
            jax.ShapeDtypeStruct((n, emb), jnp.float32),
            jax.ShapeDtypeStruct((n, 3 * emb), jnp.float32),
        ],
    )(s, v2, part)


N_WORKERS = 32
GATHER_CH = 80
SCATTER_CH = 80


def kernel(s, v, edges, r_ij, r_ij_normalized, W1, b1, W2, b2, Wr, br):
    n, emb = s.shape
    E = edges.shape[0]

    edges = edges.astype(jnp.int32)
    src = edges[:, 1]
    dst3 = edges[:, 0].reshape(N_WORKERS, (E // N_WORKERS) // SCATTER_CH,
                               SCATTER_CH)
    v2 = v.reshape(n, 3 * emb)
    r_flat = r_ij.reshape(E)
    rn_t = r_ij_normalized.T

    s_pass = _node_mlp(s, W1, b1, W2, b2, bn=400)
    g, vg = _make_gather(E, 3 * emb, N_WORKERS, GATHER_CH)(s_pass, v2, src)
    d4 = _edge_math(r_flat, rn_t, g, vg, Wr, br, be=512)
    part = _make_scatter(E, n, emb, N_WORKERS, SCATTER_CH)(d4, dst3)
    s_out, v2_out = _combine(s, v2, part, bn=400)
    return (s_out, v2_out.reshape(n, 3, emb))


# speedup vs baseline: 14.2048x; 14.2048x over previous
import functools

import jax
import jax.numpy as jnp
from jax import lax
from jax.experimental import pallas as pl
from jax.experimental.pallas import tpu as pltpu
from jax.experimental.pallas import tpu_sc as plsc

N_RBF = 20
R_CUT = 5.0


def _mlp_body(s_ref, w1_ref, b1_ref, w2_ref, b2_ref, o_ref):
    h = jnp.dot(s_ref[...], w1_ref[...], preferred_element_type=jnp.float32)
    h = h + b1_ref[...]
    h = h * jax.nn.sigmoid(h)
    o = jnp.dot(h, w2_ref[...], preferred_element_type=jnp.float32)
    o_ref[...] = o + b2_ref[...]


def _node_mlp(s, W1, b1, W2, b2, bn):
    n, emb = s.shape
    out3 = W2.shape[1]
    grid = n // bn
    return pl.pallas_call(
        _mlp_body,
        grid=(grid,),
        in_specs=[
            pl.BlockSpec((bn, emb), lambda i: (i, 0)),
            pl.BlockSpec((emb, emb), lambda i: (0, 0)),
            pl.BlockSpec((1, emb), lambda i: (0, 0)),
            pl.BlockSpec((emb, out3), lambda i: (0, 0)),
            pl.BlockSpec((1, out3), lambda i: (0, 0)),
        ],
        out_specs=pl.BlockSpec((bn, out3), lambda i: (i, 0)),
        out_shape=jax.ShapeDtypeStruct((n, out3), jnp.float32),
    )(s, W1, b1.reshape(1, emb), W2, b2.reshape(1, out3))


def _make_gather(E, D, n_workers, ch):
    ew = E // n_workers
    nch = ew // ch
    mesh = plsc.VectorSubcoreMesh(core_axis_name="c", subcore_axis_name="s")

    @functools.partial(
        pl.kernel,
        mesh=mesh,
        out_type=[
            jax.ShapeDtypeStruct((E, D), jnp.float32),
            jax.ShapeDtypeStruct((E, D), jnp.float32),
        ],
        scratch_types=[
            pltpu.VMEM((ch,), jnp.int32),
            pltpu.VMEM((ch, D), jnp.float32),
            pltpu.VMEM((ch, D), jnp.float32),
            pltpu.SemaphoreType.DMA,
            pltpu.SemaphoreType.DMA,
        ],
    )
    def gather_k(sp_hbm, v2_hbm, src_hbm, g_hbm, vg_hbm,
                 idx_v, gbuf, vbuf, sem_a, sem_b):
        wid = lax.axis_index("s") * 2 + lax.axis_index("c")
        base = wid * ew

        def body(j, carry):
            e0 = base + j * ch
            pltpu.sync_copy(src_hbm.at[pl.ds(e0, ch)], idx_v)
            cp_a = pltpu.async_copy(sp_hbm.at[idx_v], gbuf, sem_a)
            cp_b = pltpu.async_copy(v2_hbm.at[idx_v], vbuf, sem_b)
            cp_a.wait()
            cp_b.wait()
            pltpu.sync_copy(gbuf, g_hbm.at[pl.ds(e0, ch)])
            pltpu.sync_copy(vbuf, vg_hbm.at[pl.ds(e0, ch)])
            return carry

        lax.fori_loop(0, nch, body, 0)

    return gather_k


def _edge_body(r_ref, rnt_ref, g_ref, vg_ref, wr_ref, br_ref, o_ref):
    be = r_ref.shape[0]
    emb = o_ref.shape[2]
    r = r_ref[...].reshape(be, 1)
    n = lax.broadcasted_iota(jnp.int32, (be, N_RBF), 1).astype(jnp.float32) + 1.0
    rbf = jnp.sin(n * (jnp.pi / R_CUT) * r) / r
    rbf_pass = jnp.dot(rbf, wr_ref[...],
                       preferred_element_type=jnp.float32) + br_ref[...]
    f_cut = jnp.where(r <= R_CUT,
                      0.5 * (jnp.cos(jnp.pi * r / R_CUT) + 1.0), 0.0)
    pass_out = rbf_pass * f_cut * g_ref[...]
    coef = pass_out[:, 0:emb]
    ds = pass_out[:, emb:2 * emb]
    rep = pass_out[:, 2 * emb:3 * emb]
    o_ref[0] = ds
    for d in range(3):
        rn_d = rnt_ref[d, :].reshape(be, 1)
        o_ref[d + 1] = vg_ref[:, d * emb:(d + 1) * emb] * coef + rn_d * rep


def _edge_math(r_flat, rn_t, g, vg, Wr, br, be):
    E = r_flat.shape[0]
    emb = g.shape[1] // 3
    grid = E // be
    return pl.pallas_call(
        _edge_body,
        grid=(grid,),
        in_specs=[
            pl.BlockSpec((be,), lambda i: (i,)),
            pl.BlockSpec((3, be), lambda i: (0, i)),
            pl.BlockSpec((be, 3 * emb), lambda i: (i, 0)),
            pl.BlockSpec((be, 3 * emb), lambda i: (i, 0)),
            pl.BlockSpec((N_RBF, 3 * emb), lambda i: (0, 0)),
            pl.BlockSpec((1, 3 * emb), lambda i: (0, 0)),
        ],
        out_specs=pl.BlockSpec((4, be, emb), lambda i: (0, i, 0)),
        out_shape=jax.ShapeDtypeStruct((4, E, emb), jnp.float32),
    )(r_flat, rn_t, g, vg, Wr, br.reshape(1, 3 * emb))


def _make_scatter(E, N, emb, n_workers, ch):
    ew = E // n_workers
    nch = ew // ch
    n_sub = 16
    zch = 80
    n_rowch = N // zch
    rounds = (n_rowch + n_sub - 1) // n_sub
    mesh = plsc.VectorSubcoreMesh(core_axis_name="c", subcore_axis_name="s")

    @functools.partial(
        pl.kernel,
        mesh=mesh,
        out_type=jax.ShapeDtypeStruct((2, 4, N, emb), jnp.float32),
        scratch_types=[
            pltpu.VMEM((nch, ch), jnp.int32),
            pltpu.VMEM((ch, emb), jnp.float32),
            pltpu.VMEM((zch, emb), jnp.float32),
            pltpu.VMEM((zch, emb), jnp.float32),
            pltpu.VMEM_SHARED((N, emb), jnp.float32),
        ],
    )
    def scatter_k(d4_hbm, dst3_hbm, part_hbm,
                  idxmat, dbuf, zbuf, obuf, accum):
        cid = lax.axis_index("c")
        sid = lax.axis_index("s")
        wid = sid * 2 + cid
        base = wid * ew

        zero = jnp.zeros((16,), jnp.float32)
        def zrow(i, carry):
            for q in range(emb // 16):
                zbuf[i, pl.ds(q * 16, 16)] = zero
            return carry
        lax.fori_loop(0, zch, zrow, 0)

        pltpu.sync_copy(dst3_hbm.at[wid], idxmat)

        for p in range(4):
            for q in range(rounds):
                rc = sid + q * n_sub

                @pl.when(rc < n_rowch)
                def _zero():
                    pltpu.sync_copy(zbuf, accum.at[pl.ds(rc * zch, zch)])
            plsc.subcore_barrier()

            def body(j, carry):
                e0 = base + j * ch
                pltpu.sync_copy(d4_hbm.at[p, pl.ds(e0, ch)], dbuf)
                pltpu.sync_copy(dbuf, accum.at[idxmat.at[j]], add=True)
                return carry
            lax.fori_loop(0, nch, body, 0)
            plsc.subcore_barrier()

            for q in range(rounds):
                rc = sid + q * n_sub

                @pl.when(rc < n_rowch)
                def _dump():
                    r0 = rc * zch
                    pltpu.sync_copy(accum.at[pl.ds(r0, zch)], obuf)
                    pltpu.sync_copy(obuf,
                                    part_hbm.at[cid, p, pl.ds(r0, zch)])
            plsc.subcore_barrier()

    return scatter_k


def _combine_body(s_ref, v2_ref, part_ref, so_ref, vo_ref):
    so_ref[...] = s_ref[...] + part_ref[0, 0] + part_ref[1, 0]
    dv = [part_ref[0, 1 + d] + part_ref[1, 1 + d] for d in range(3)]
    vo_ref[...] = v2_ref[...] + jnp.concatenate(dv, axis=1)


def _combine(s, v2, part, bn):
    n, emb = s.shape
    grid = n // bn
    return pl.pallas_call(
        _combine_body,
        grid=(grid,),
        in_specs=[
            pl.BlockSpec((bn, emb), lambda i: (i, 0)),
            pl.BlockSpec((bn, 3 * emb), lambda i: (i, 0)),
            pl.BlockSpec((2, 4, bn, emb), lambda i: (0, 0, i, 0)),
        ],
        out_specs=[
            pl.BlockSpec((bn, emb), lambda i: (i, 0)),
            pl.BlockSpec((bn, 3 * emb), lambda i: (i, 0)),
        ],
        out_shape=[
            jax.ShapeDtypeStruct((n, emb), jnp.float32),
            jax.ShapeDtypeStruct((n, 3 * emb), jnp.float32),
        ],
    )(s, v2, part)


N_WORKERS = 32
GATHER_CH = 80
SCATTER_CH = 80


def kernel(s, v, edges, r_ij, r_ij_normalized, W1, b1, W2, b2, Wr, br):
    n, emb = s.shape
    E = edges.shape[0]

    edges = edges.astype(jnp.int32)
    src = edges[:, 1]
    dst3 = edges[:, 0].reshape(N_WORKERS, (E // N_WORKERS) // SCATTER_CH,
                               SCATTER_CH)
    v2 = v.reshape(n, 3 * emb)
    r_flat = r_ij.reshape(E)
    rn_t = r_ij_normalized.T

    s_pass = _node_mlp(s, W1, b1, W2, b2, bn=400)
    g, vg = _make_gather(E, 3 * emb, N_WORKERS, GATHER_CH)(s_pass, v2, src)
    d4 = _edge_math(r_flat, rn_t, g, vg, Wr, br, be=512)
    part = _make_scatter(E, n, emb, N_WORKERS, SCATTER_CH)(d4, dst3)
    s_out, v2_out = _combine(s, v2, part, bn=400)
    return (s_out, v2_out.reshape(n, 3, emb))


# speedup vs baseline: 15.9555x; 1.1232x over previous
import functools

import jax
import jax.numpy as jnp
from jax import lax
from jax.experimental import pallas as pl
from jax.experimental.pallas import tpu as pltpu
from jax.experimental.pallas import tpu_sc as plsc

N_RBF = 20
R_CUT = 5.0


def _mlp_body(s_ref, w1_ref, b1_ref, w2_ref, b2_ref, o_ref):
    h = jnp.dot(s_ref[...], w1_ref[...], preferred_element_type=jnp.float32)
    h = h + b1_ref[...]
    h = h * jax.nn.sigmoid(h)
    o = jnp.dot(h, w2_ref[...], preferred_element_type=jnp.float32)
    o_ref[...] = o + b2_ref[...]


def _node_mlp(s, W1, b1, W2, b2, bn):
    n, emb = s.shape
    out3 = W2.shape[1]
    grid = n // bn
    return pl.pallas_call(
        _mlp_body,
        grid=(grid,),
        in_specs=[
            pl.BlockSpec((bn, emb), lambda i: (i, 0)),
            pl.BlockSpec((emb, emb), lambda i: (0, 0)),
            pl.BlockSpec((1, emb), lambda i: (0, 0)),
            pl.BlockSpec((emb, out3), lambda i: (0, 0)),
            pl.BlockSpec((1, out3), lambda i: (0, 0)),
        ],
        out_specs=pl.BlockSpec((bn, out3), lambda i: (i, 0)),
        out_shape=jax.ShapeDtypeStruct((n, out3), jnp.float32),
    )(s, W1, b1.reshape(1, emb), W2, b2.reshape(1, out3))


def _make_gather(E, D, n_workers, ch):
    ew = E // n_workers
    nch = ew // ch
    assert nch % 2 == 0
    mesh = plsc.VectorSubcoreMesh(core_axis_name="c", subcore_axis_name="s")

    @functools.partial(
        pl.kernel,
        mesh=mesh,
        out_type=[
            jax.ShapeDtypeStruct((E, D), jnp.float32),
            jax.ShapeDtypeStruct((E, D), jnp.float32),
        ],
        scratch_types=[
            pltpu.VMEM((2, ch), jnp.int32),
            pltpu.VMEM((ch, D), jnp.float32),
            pltpu.VMEM((ch, D), jnp.float32),
            pltpu.VMEM((ch, D), jnp.float32),
            pltpu.VMEM((ch, D), jnp.float32),
            pltpu.SemaphoreType.DMA,
            pltpu.SemaphoreType.DMA,
            pltpu.SemaphoreType.DMA,
            pltpu.SemaphoreType.DMA,
        ],
    )
    def gather_k(sp_hbm, v2_hbm, src_hbm, g_hbm, vg_hbm,
                 idx_v, gb0, gb1, vb0, vb1, sg0, sg1, sv0, sv1):
        wid = lax.axis_index("s") * 2 + lax.axis_index("c")
        base = wid * ew
        gbufs = (gb0, gb1)
        vbufs = (vb0, vb1)
        gsems = (sg0, sg1)
        vsems = (sv0, sv1)

        def start(j, b):
            pltpu.sync_copy(src_hbm.at[pl.ds(base + j * ch, ch)],
                            idx_v.at[b])
            pltpu.async_copy(sp_hbm.at[idx_v.at[b]], gbufs[b], gsems[b])
            pltpu.async_copy(v2_hbm.at[idx_v.at[b]], vbufs[b], vsems[b])

        def finish(j, b):
            pltpu.make_async_copy(sp_hbm.at[idx_v.at[b]], gbufs[b],
                                  gsems[b]).wait()
            pltpu.make_async_copy(v2_hbm.at[idx_v.at[b]], vbufs[b],
                                  vsems[b]).wait()
            e0 = base + j * ch
            pltpu.sync_copy(gbufs[b], g_hbm.at[pl.ds(e0, ch)])
            pltpu.sync_copy(vbufs[b], vg_hbm.at[pl.ds(e0, ch)])

        start(0, 0)

        def body(jj, carry):
            for b in range(2):
                j = jj * 2 + b
                nxt = j + 1

                @pl.when(nxt < nch)
                def _():
                    start(nxt, (b + 1) % 2)
                finish(j, b)
            return carry

        lax.fori_loop(0, nch // 2, body, 0)

    return gather_k


def _edge_body(r_ref, rnt_ref, g_ref, vg_ref, wr_ref, br_ref, o_ref):
    be = r_ref.shape[0]
    emb = o_ref.shape[2]
    r = r_ref[...].reshape(be, 1)
    rr = r_ref[...].reshape(1, be)
    n_t = (lax.broadcasted_iota(jnp.int32, (N_RBF, be), 0)
           .astype(jnp.float32) + 1.0)
    rbf_t = jnp.sin(n_t * (jnp.pi / R_CUT) * rr) / rr
    rbf_pass = lax.dot_general(
        rbf_t, wr_ref[...], (((0,), (0,)), ((), ())),
        preferred_element_type=jnp.float32) + br_ref[...]
    f_cut = jnp.where(r <= R_CUT,
                      0.5 * (jnp.cos(jnp.pi * r / R_CUT) + 1.0), 0.0)
    pass_out = rbf_pass * f_cut * g_ref[...]
    coef = pass_out[:, 0:emb]
    ds = pass_out[:, emb:2 * emb]
    rep = pass_out[:, 2 * emb:3 * emb]
    o_ref[0] = ds
    for d in range(3):
        rn_d = rnt_ref[d, :].reshape(be, 1)
        o_ref[d + 1] = vg_ref[:, d * emb:(d + 1) * emb] * coef + rn_d * rep


def _edge_math(r_flat, rn_t, g, vg, Wr, br, be):
    E = r_flat.shape[0]
    emb = g.shape[1] // 3
    grid = E // be
    return pl.pallas_call(
        _edge_body,
        grid=(grid,),
        in_specs=[
            pl.BlockSpec((be,), lambda i: (i,)),
            pl.BlockSpec((3, be), lambda i: (0, i)),
            pl.BlockSpec((be, 3 * emb), lambda i: (i, 0)),
            pl.BlockSpec((be, 3 * emb), lambda i: (i, 0)),
            pl.BlockSpec((N_RBF, 3 * emb), lambda i: (0, 0)),
            pl.BlockSpec((1, 3 * emb), lambda i: (0, 0)),
        ],
        out_specs=pl.BlockSpec((4, be, emb), lambda i: (0, i, 0)),
        out_shape=jax.ShapeDtypeStruct((4, E, emb), jnp.float32),
    )(r_flat, rn_t, g, vg, Wr, br.reshape(1, 3 * emb))


def _make_scatter(E, N, emb, n_workers, ch):
    ew = E // n_workers
    nch = ew // ch
    n_sub = 16
    zch = 80
    n_rowch = N // zch
    rounds = (n_rowch + n_sub - 1) // n_sub
    mesh = plsc.VectorSubcoreMesh(core_axis_name="c", subcore_axis_name="s")

    @functools.partial(
        pl.kernel,
        mesh=mesh,
        out_type=jax.ShapeDtypeStruct((2, 4, N, emb), jnp.float32),
        scratch_types=[
            pltpu.VMEM((nch, ch), jnp.int32),
            pltpu.VMEM((ch, emb), jnp.float32),
            pltpu.VMEM((ch, emb), jnp.float32),
            pltpu.VMEM((zch, emb), jnp.float32),
            pltpu.VMEM((zch, emb), jnp.float32),
            pltpu.SemaphoreType.DMA,
            pltpu.SemaphoreType.DMA,
            pltpu.VMEM_SHARED((N, emb), jnp.float32),
        ],
    )
    def scatter_k(d4_hbm, dst3_hbm, part_hbm,
                  idxmat, db0, db1, zbuf, obuf, sd0, sd1, accum):
        cid = lax.axis_index("c")
        sid = lax.axis_index("s")
        wid = sid * 2 + cid
        base = wid * ew
        dbufs = (db0, db1)
        dsems = (sd0, sd1)

        zero = jnp.zeros((16,), jnp.float32)
        def zrow(i, carry):
            for q in range(emb // 16):
                zbuf[i, pl.ds(q * 16, 16)] = zero
            return carry
        lax.fori_loop(0, zch, zrow, 0)

        pltpu.sync_copy(dst3_hbm.at[wid], idxmat)

        for p in range(4):
            for q in range(rounds):
                rc = sid + q * n_sub

                @pl.when(rc < n_rowch)
                def _zero():
                    pltpu.sync_copy(zbuf, accum.at[pl.ds(rc * zch, zch)])
            plsc.subcore_barrier()

            def body(j, carry):
                e0 = base + j * ch
                pltpu.sync_copy(d4_hbm.at[p, pl.ds(e0, ch)], dbufs[0])
                pltpu.sync_copy(dbufs[0], accum.at[idxmat.at[j]],
                                add=True)
                return carry
            lax.fori_loop(0, nch, body, 0)
            plsc.subcore_barrier()

            for q in range(rounds):
                rc = sid + q * n_sub

                @pl.when(rc < n_rowch)
                def _dump():
                    r0 = rc * zch
                    pltpu.sync_copy(accum.at[pl.ds(r0, zch)], obuf)
                    pltpu.sync_copy(obuf,
                                    part_hbm.at[cid, p, pl.ds(r0, zch)])
            plsc.subcore_barrier()

    return scatter_k


def _combine_body(s_ref, v2_ref, part_ref, so_ref, vo_ref):
    so_ref[...] = s_ref[...] + part_ref[0, 0] + part_ref[1, 0]
    dv = [part_ref[0, 1 + d] + part_ref[1, 1 + d] for d in range(3)]
    vo_ref[...] = v2_ref[...] + jnp.concatenate(dv, axis=1)


def _combine(s, v2, part, bn):
    n, emb = s.shape
    grid = n // bn
    return pl.pallas_call(
        _combine_body,
        grid=(grid,),
        in_specs=[
            pl.BlockSpec((bn, emb), lambda i: (i, 0)),
            pl.BlockSpec((bn, 3 * emb), lambda i: (i, 0)),
            pl.BlockSpec((2, 4, bn, emb), lambda i: (0, 0, i, 0)),
        ],
        out_specs=[
            pl.BlockSpec((bn, emb), lambda i: (i, 0)),
            pl.BlockSpec((bn, 3 * emb), lambda i: (i, 0)),
        ],
        out_shape=[
            jax.ShapeDtypeStruct((n, emb), jnp.float32),
            jax.ShapeDtypeStruct((n, 3 * emb), jnp.float32),
        ],
    )(s, v2, part)


N_WORKERS = 32
GATHER_CH = 40
SCATTER_CH = 80


def kernel(s, v, edges, r_ij, r_ij_normalized, W1, b1, W2, b2, Wr, br):
    n, emb = s.shape
    E = edges.shape[0]

    edges = edges.astype(jnp.int32)
    src = edges[:, 1]
    dst3 = edges[:, 0].reshape(N_WORKERS, (E // N_WORKERS) // SCATTER_CH,
                               SCATTER_CH)
    v2 = v.reshape(n, 3 * emb)
    r_flat = r_ij.reshape(E)
    rn_t = r_ij_normalized.T

    s_pass = _node_mlp(s, W1, b1, W2, b2, bn=400)
    g, vg = _make_gather(E, 3 * emb, N_WORKERS, GATHER_CH)(s_pass, v2, src)
    d4 = _edge_math(r_flat, rn_t, g, vg, Wr, br, be=512)
    part = _make_scatter(E, n, emb, N_WORKERS, SCATTER_CH)(d4, dst3)
    s_out, v2_out = _combine(s, v2, part, bn=400)
    return (s_out, v2_out.reshape(n, 3, emb))


# speedup vs baseline: 18.3112x; 1.1476x over previous
import functools

import jax
import jax.numpy as jnp
from jax import lax
from jax.experimental import pallas as pl
from jax.experimental.pallas import tpu as pltpu
from jax.experimental.pallas import tpu_sc as plsc

N_RBF = 20
R_CUT = 5.0


def _mlp_body(s_ref, w1_ref, b1_ref, w2_ref, b2_ref, o_ref):
    h = jnp.dot(s_ref[...], w1_ref[...], preferred_element_type=jnp.float32)
    h = h + b1_ref[...]
    h = h * jax.nn.sigmoid(h)
    o = jnp.dot(h, w2_ref[...], preferred_element_type=jnp.float32)
    o_ref[...] = o + b2_ref[...]


def _node_mlp(s, W1, b1, W2, b2, bn):
    n, emb = s.shape
    out3 = W2.shape[1]
    grid = n // bn
    return pl.pallas_call(
        _mlp_body,
        grid=(grid,),
        in_specs=[
            pl.BlockSpec((bn, emb), lambda i: (i, 0)),
            pl.BlockSpec((emb, emb), lambda i: (0, 0)),
            pl.BlockSpec((1, emb), lambda i: (0, 0)),
            pl.BlockSpec((emb, out3), lambda i: (0, 0)),
            pl.BlockSpec((1, out3), lambda i: (0, 0)),
        ],
        out_specs=pl.BlockSpec((bn, out3), lambda i: (i, 0)),
        out_shape=jax.ShapeDtypeStruct((n, out3), jnp.float32),
    )(s, W1, b1.reshape(1, emb), W2, b2.reshape(1, out3))


def _make_gather(E, D, n_workers, ch):
    ew = E // n_workers
    nch = ew // ch
    assert nch % 2 == 0
    mesh = plsc.VectorSubcoreMesh(core_axis_name="c", subcore_axis_name="s")

    @functools.partial(
        pl.kernel,
        mesh=mesh,
        out_type=[
            jax.ShapeDtypeStruct((E, D), jnp.float32),
            jax.ShapeDtypeStruct((E, D), jnp.float32),
        ],
        scratch_types=[
            pltpu.VMEM((2, ch), jnp.int32),
            pltpu.VMEM((ch, D), jnp.float32),
            pltpu.VMEM((ch, D), jnp.float32),
            pltpu.VMEM((ch, D), jnp.float32),
            pltpu.VMEM((ch, D), jnp.float32),
            pltpu.SemaphoreType.DMA,
            pltpu.SemaphoreType.DMA,
            pltpu.SemaphoreType.DMA,
            pltpu.SemaphoreType.DMA,
        ],
    )
    def gather_k(sp_hbm, v2_hbm, src_hbm, g_hbm, vg_hbm,
                 idx_v, gb0, gb1, vb0, vb1, sg0, sg1, sv0, sv1):
        wid = lax.axis_index("s") * 2 + lax.axis_index("c")
        base = wid * ew
        gbufs = (gb0, gb1)
        vbufs = (vb0, vb1)
        gsems = (sg0, sg1)
        vsems = (sv0, sv1)

        def start(j, b):
            pltpu.sync_copy(src_hbm.at[pl.ds(base + j * ch, ch)],
                            idx_v.at[b])
            pltpu.async_copy(sp_hbm.at[idx_v.at[b]], gbufs[b], gsems[b])
            pltpu.async_copy(v2_hbm.at[idx_v.at[b]], vbufs[b], vsems[b])

        def finish(j, b):
            pltpu.make_async_copy(sp_hbm.at[idx_v.at[b]], gbufs[b],
                                  gsems[b]).wait()
            pltpu.make_async_copy(v2_hbm.at[idx_v.at[b]], vbufs[b],
                                  vsems[b]).wait()
            e0 = base + j * ch
            pltpu.sync_copy(gbufs[b], g_hbm.at[pl.ds(e0, ch)])
            pltpu.sync_copy(vbufs[b], vg_hbm.at[pl.ds(e0, ch)])

        start(0, 0)

        def body(jj, carry):
            for b in range(2):
                j = jj * 2 + b
                nxt = j + 1

                @pl.when(nxt < nch)
                def _():
                    start(nxt, (b + 1) % 2)
                finish(j, b)
            return carry

        lax.fori_loop(0, nch // 2, body, 0)

    return gather_k


def _edge_body(r_ref, rnt_ref, g_ref, vg_ref, wr_ref, br_ref, o_ref):
    be = r_ref.shape[0]
    emb = o_ref.shape[2]
    r = r_ref[...].reshape(be, 1)
    rr = r_ref[...].reshape(1, be)
    n_t = (lax.broadcasted_iota(jnp.int32, (N_RBF, be), 0)
           .astype(jnp.float32) + 1.0)
    rbf_t = jnp.sin(n_t * (jnp.pi / R_CUT) * rr) / rr
    rbf_pass = lax.dot_general(
        rbf_t, wr_ref[...], (((0,), (0,)), ((), ())),
        preferred_element_type=jnp.float32) + br_ref[...]
    f_cut = jnp.where(r <= R_CUT,
                      0.5 * (jnp.cos(jnp.pi * r / R_CUT) + 1.0), 0.0)
    pass_out = rbf_pass * f_cut * g_ref[...]
    coef = pass_out[:, 0:emb]
    ds = pass_out[:, emb:2 * emb]
    rep = pass_out[:, 2 * emb:3 * emb]
    o_ref[0] = ds
    for d in range(3):
        rn_d = rnt_ref[d, :].reshape(be, 1)
        o_ref[d + 1] = vg_ref[:, d * emb:(d + 1) * emb] * coef + rn_d * rep


def _edge_math(r_flat, rn_t, g, vg, Wr, br, be):
    E = r_flat.shape[0]
    emb = g.shape[1] // 3
    grid = E // be
    return pl.pallas_call(
        _edge_body,
        grid=(grid,),
        in_specs=[
            pl.BlockSpec((be,), lambda i: (i,)),
            pl.BlockSpec((3, be), lambda i: (0, i)),
            pl.BlockSpec((be, 3 * emb), lambda i: (i, 0)),
            pl.BlockSpec((be, 3 * emb), lambda i: (i, 0)),
            pl.BlockSpec((N_RBF, 3 * emb), lambda i: (0, 0)),
            pl.BlockSpec((1, 3 * emb), lambda i: (0, 0)),
        ],
        out_specs=pl.BlockSpec((4, be, emb), lambda i: (0, i, 0)),
        out_shape=jax.ShapeDtypeStruct((4, E, emb), jnp.float32),
    )(r_flat, rn_t, g, vg, Wr, br.reshape(1, 3 * emb))


def _make_scatter(E, N, emb, n_workers, ch):
    ew = E // n_workers
    nch = ew // ch
    n_sub = 16
    zch = 80
    n_rowch = N // zch
    rounds = (n_rowch + n_sub - 1) // n_sub
    mesh = plsc.VectorSubcoreMesh(core_axis_name="c", subcore_axis_name="s")

    @functools.partial(
        pl.kernel,
        mesh=mesh,
        out_type=jax.ShapeDtypeStruct((2, 4, N, emb), jnp.float32),
        scratch_types=[
            pltpu.VMEM((nch, ch), jnp.int32),
            pltpu.VMEM((ch, emb), jnp.float32),
            pltpu.VMEM((ch, emb), jnp.float32),
            pltpu.VMEM((zch, emb), jnp.float32),
            pltpu.SemaphoreType.DMA,
            pltpu.SemaphoreType.DMA,
            pltpu.VMEM_SHARED((N, emb), jnp.float32),
        ],
    )
    def scatter_k(d4_hbm, dst3_hbm, part_hbm,
                  idxmat, db0, db1, obuf, sd0, sd1, accum):
        cid = lax.axis_index("c")
        sid = lax.axis_index("s")
        wid = sid * 2 + cid
        base = wid * ew
        dbufs = (db0, db1)
        dsems = (sd0, sd1)

        pltpu.sync_copy(dst3_hbm.at[wid], idxmat)

        zero = jnp.zeros((16,), jnp.float32)

        for p in range(4):
            def zrow(i, carry):
                for q in range(emb // 16):
                    obuf[i, pl.ds(q * 16, 16)] = zero
                return carry
            lax.fori_loop(0, zch, zrow, 0)

            for q in range(rounds):
                rc = sid + q * n_sub

                @pl.when(rc < n_rowch)
                def _zero():
                    pltpu.sync_copy(obuf, accum.at[pl.ds(rc * zch, zch)])
            plsc.subcore_barrier()

            def start(j, b):
                pltpu.async_copy(d4_hbm.at[p, pl.ds(base + j * ch, ch)],
                                 dbufs[b], dsems[b])

            def drain(j, b):
                pltpu.make_async_copy(
                    d4_hbm.at[p, pl.ds(base + j * ch, ch)],
                    dbufs[b], dsems[b]).wait()
                pltpu.sync_copy(dbufs[b], accum.at[idxmat.at[j]],
                                add=True)

            start(0, 0)

            def body(jj, carry):
                for b in range(2):
                    j = jj * 2 + b
                    nxt = j + 1

                    @pl.when(nxt < nch)
                    def _():
                        start(nxt, (b + 1) % 2)

                    @pl.when(j < nch)
                    def _():
                        drain(j, b)
                return carry
            lax.fori_loop(0, (nch + 1) // 2, body, 0)
            plsc.subcore_barrier()

            for q in range(rounds):
                rc = sid + q * n_sub

                @pl.when(rc < n_rowch)
                def _dump():
                    r0 = rc * zch
                    pltpu.sync_copy(accum.at[pl.ds(r0, zch)], obuf)
                    pltpu.sync_copy(obuf,
                                    part_hbm.at[cid, p, pl.ds(r0, zch)])
            plsc.subcore_barrier()

    return scatter_k


def _combine_body(s_ref, v2_ref, part_ref, so_ref, vo_ref):
    so_ref[...] = s_ref[...] + part_ref[0, 0] + part_ref[1, 0]
    dv = [part_ref[0, 1 + d] + part_ref[1, 1 + d] for d in range(3)]
    vo_ref[...] = v2_ref[...] + jnp.concatenate(dv, axis=1)


def _combine(s, v2, part, bn):
    n, emb = s.shape
    grid = n // bn
    return pl.pallas_call(
        _combine_body,
        grid=(grid,),
        in_specs=[
            pl.BlockSpec((bn, emb), lambda i: (i, 0)),
            pl.BlockSpec((bn, 3 * emb), lambda i: (i, 0)),
            pl.BlockSpec((2, 4, bn, emb), lambda i: (0, 0, i, 0)),
        ],
        out_specs=[
            pl.BlockSpec((bn, emb), lambda i: (i, 0)),
            pl.BlockSpec((bn, 3 * emb), lambda i: (i, 0)),
        ],
        out_shape=[
            jax.ShapeDtypeStruct((n, emb), jnp.float32),
            jax.ShapeDtypeStruct((n, 3 * emb), jnp.float32),
        ],
    )(s, v2, part)


N_WORKERS = 32
GATHER_CH = 40
SCATTER_CH = 80


def kernel(s, v, edges, r_ij, r_ij_normalized, W1, b1, W2, b2, Wr, br):
    n, emb = s.shape
    E = edges.shape[0]

    edges = edges.astype(jnp.int32)
    src = edges[:, 1]
    dst3 = edges[:, 0].reshape(N_WORKERS, (E // N_WORKERS) // SCATTER_CH,
                               SCATTER_CH)
    v2 = v.reshape(n, 3 * emb)
    r_flat = r_ij.reshape(E)
    rn_t = r_ij_normalized.T

    s_pass = _node_mlp(s, W1, b1, W2, b2, bn=400)
    g, vg = _make_gather(E, 3 * emb, N_WORKERS, GATHER_CH)(s_pass, v2, src)
    d4 = _edge_math(r_flat, rn_t, g, vg, Wr, br, be=512)
    part = _make_scatter(E, n, emb, N_WORKERS, SCATTER_CH)(d4, dst3)
    s_out, v2_out = _combine(s, v2, part, bn=400)
    return (s_out, v2_out.reshape(n, 3, emb))


# speedup vs baseline: 20.7076x; 1.1309x over previous
import functools

import jax
import jax.numpy as jnp
from jax import lax
from jax.experimental import pallas as pl
from jax.experimental.pallas import tpu as pltpu
from jax.experimental.pallas import tpu_sc as plsc

N_RBF = 20
R_CUT = 5.0


def _mlp_body(s_ref, w1_ref, b1_ref, w2_ref, b2_ref, o_ref):
    h = jnp.dot(s_ref[...], w1_ref[...], preferred_element_type=jnp.float32)
    h = h + b1_ref[...]
    h = h * jax.nn.sigmoid(h)
    o = jnp.dot(h, w2_ref[...], preferred_element_type=jnp.float32)
    o_ref[...] = o + b2_ref[...]


def _node_mlp(s, W1, b1, W2, b2, bn):
    n, emb = s.shape
    out3 = W2.shape[1]
    grid = n // bn
    return pl.pallas_call(
        _mlp_body,
        grid=(grid,),
        in_specs=[
            pl.BlockSpec((bn, emb), lambda i: (i, 0)),
            pl.BlockSpec((emb, emb), lambda i: (0, 0)),
            pl.BlockSpec((1, emb), lambda i: (0, 0)),
            pl.BlockSpec((emb, out3), lambda i: (0, 0)),
            pl.BlockSpec((1, out3), lambda i: (0, 0)),
        ],
        out_specs=pl.BlockSpec((bn, out3), lambda i: (i, 0)),
        out_shape=jax.ShapeDtypeStruct((n, out3), jnp.float32),
    )(s, W1, b1.reshape(1, emb), W2, b2.reshape(1, out3))


def _make_gather(E, D, n_workers, ch):
    ew = E // n_workers
    nch = ew // ch
    assert nch % 2 == 0
    mesh = plsc.VectorSubcoreMesh(core_axis_name="c", subcore_axis_name="s")

    @functools.partial(
        pl.kernel,
        mesh=mesh,
        out_type=[
            jax.ShapeDtypeStruct((E, D), jnp.float32),
            jax.ShapeDtypeStruct((E, D), jnp.float32),
        ],
        scratch_types=[
            pltpu.VMEM((2, ch), jnp.int32),
            pltpu.VMEM((ch, D), jnp.float32),
            pltpu.VMEM((ch, D), jnp.float32),
            pltpu.VMEM((ch, D), jnp.float32),
            pltpu.VMEM((ch, D), jnp.float32),
            pltpu.SemaphoreType.DMA,
            pltpu.SemaphoreType.DMA,
            pltpu.SemaphoreType.DMA,
            pltpu.SemaphoreType.DMA,
        ],
    )
    def gather_k(sp_hbm, v2_hbm, src_hbm, g_hbm, vg_hbm,
                 idx_v, gb0, gb1, vb0, vb1, sg0, sg1, sv0, sv1):
        wid = lax.axis_index("s") * 2 + lax.axis_index("c")
        base = wid * ew
        gbufs = (gb0, gb1)
        vbufs = (vb0, vb1)
        gsems = (sg0, sg1)
        vsems = (sv0, sv1)

        def start(j, b):
            pltpu.sync_copy(src_hbm.at[pl.ds(base + j * ch, ch)],
                            idx_v.at[b])
            pltpu.async_copy(sp_hbm.at[idx_v.at[b]], gbufs[b], gsems[b])
            pltpu.async_copy(v2_hbm.at[idx_v.at[b]], vbufs[b], vsems[b])

        def finish(j, b):
            pltpu.make_async_copy(sp_hbm.at[idx_v.at[b]], gbufs[b],
                                  gsems[b]).wait()
            pltpu.make_async_copy(v2_hbm.at[idx_v.at[b]], vbufs[b],
                                  vsems[b]).wait()
            e0 = base + j * ch
            pltpu.sync_copy(gbufs[b], g_hbm.at[pl.ds(e0, ch)])
            pltpu.sync_copy(vbufs[b], vg_hbm.at[pl.ds(e0, ch)])

        start(0, 0)

        def body(jj, carry):
            for b in range(2):
                j = jj * 2 + b
                nxt = j + 1

                @pl.when(nxt < nch)
                def _():
                    start(nxt, (b + 1) % 2)
                finish(j, b)
            return carry

        lax.fori_loop(0, nch // 2, body, 0)

    return gather_k


def _fast_sin(x):
    m = x * (1.0 / jnp.pi)
    k = jnp.floor(m + 0.5)
    y = (m - k) * jnp.pi
    y2 = y * y
    p = jnp.float32(-2.5052108e-08)
    p = p * y2 + 2.7557319e-06
    p = p * y2 - 1.9841270e-04
    p = p * y2 + 8.3333333e-03
    p = p * y2 - 1.6666667e-01
    s = y + y * (y2 * p)
    odd = k - 2.0 * jnp.floor(k * 0.5)
    return s * (1.0 - 2.0 * odd)


def _edge_body(r_ref, rnt_ref, nc_ref, g_ref, vg_ref, wra_ref, o_ref):
    be = r_ref.shape[0]
    emb = o_ref.shape[2]
    rr = r_ref[...].reshape(1, be)
    sins = _fast_sin(nc_ref[...] * rr)
    cosx = _fast_sin(jnp.pi / 2 + (jnp.pi / R_CUT) * rr)
    f_cut = jnp.where(rr <= R_CUT, 0.5 * (cosx + 1.0), 0.0)
    aug = jnp.concatenate([sins, rr], axis=0)
    aug = aug * (f_cut / rr)
    rbf_fcut = lax.dot_general(
        aug, wra_ref[...], (((0,), (0,)), ((), ())),
        preferred_element_type=jnp.float32)
    pass_out = rbf_fcut * g_ref[...]
    coef = pass_out[:, 0:emb]
    ds = pass_out[:, emb:2 * emb]
    rep = pass_out[:, 2 * emb:3 * emb]
    o_ref[0] = ds
    for d in range(3):
        rn_d = rnt_ref[d, :].reshape(be, 1)
        o_ref[d + 1] = vg_ref[:, d * emb:(d + 1) * emb] * coef + rn_d * rep


def _edge_math(r_flat, rn_t, g, vg, Wr, br, be):
    E = r_flat.shape[0]
    emb = g.shape[1] // 3
    grid = E // be
    nc = (jnp.arange(1, N_RBF + 1, dtype=jnp.float32)
          * (jnp.pi / R_CUT)).reshape(N_RBF, 1)
    wr_aug = jnp.concatenate([Wr, br.reshape(1, 3 * emb)], axis=0)
    return pl.pallas_call(
        _edge_body,
        grid=(grid,),
        in_specs=[
            pl.BlockSpec((be,), lambda i: (i,)),
            pl.BlockSpec((3, be), lambda i: (0, i)),
            pl.BlockSpec((N_RBF, 1), lambda i: (0, 0)),
            pl.BlockSpec((be, 3 * emb), lambda i: (i, 0)),
            pl.BlockSpec((be, 3 * emb), lambda i: (i, 0)),
            pl.BlockSpec((N_RBF + 1, 3 * emb), lambda i: (0, 0)),
        ],
        out_specs=pl.BlockSpec((4, be, emb), lambda i: (0, i, 0)),
        out_shape=jax.ShapeDtypeStruct((4, E, emb), jnp.float32),
    )(r_flat, rn_t, nc, g, vg, wr_aug)


def _make_scatter(E, N, emb, n_workers, ch):
    ew = E // n_workers
    nch = ew // ch
    n_sub = 16
    zch = 80
    n_rowch = N // zch
    rounds = (n_rowch + n_sub - 1) // n_sub
    mesh = plsc.VectorSubcoreMesh(core_axis_name="c", subcore_axis_name="s")

    @functools.partial(
        pl.kernel,
        mesh=mesh,
        out_type=jax.ShapeDtypeStruct((2, 4, N, emb), jnp.float32),
        scratch_types=[
            pltpu.VMEM((nch, ch), jnp.int32),
            pltpu.VMEM((ch, emb), jnp.float32),
            pltpu.VMEM((ch, emb), jnp.float32),
            pltpu.VMEM((zch, emb), jnp.float32),
            pltpu.SemaphoreType.DMA,
            pltpu.SemaphoreType.DMA,
            pltpu.VMEM_SHARED((N, emb), jnp.float32),
        ],
    )
    def scatter_k(d4_hbm, dst3_hbm, part_hbm,
                  idxmat, db0, db1, obuf, sd0, sd1, accum):
        cid = lax.axis_index("c")
        sid = lax.axis_index("s")
        wid = sid * 2 + cid
        base = wid * ew
        dbufs = (db0, db1)
        dsems = (sd0, sd1)

        pltpu.sync_copy(dst3_hbm.at[wid], idxmat)

        zero = jnp.zeros((16,), jnp.float32)

        for p in range(4):
            def zrow(i, carry):
                for q in range(emb // 16):
                    obuf[i, pl.ds(q * 16, 16)] = zero
                return carry
            lax.fori_loop(0, zch, zrow, 0)

            for q in range(rounds):
                rc = sid + q * n_sub

                @pl.when(rc < n_rowch)
                def _zero():
                    pltpu.sync_copy(obuf, accum.at[pl.ds(rc * zch, zch)])
            plsc.subcore_barrier()

            def start(j, b):
                pltpu.async_copy(d4_hbm.at[p, pl.ds(base + j * ch, ch)],
                                 dbufs[b], dsems[b])

            def drain(j, b):
                pltpu.make_async_copy(
                    d4_hbm.at[p, pl.ds(base + j * ch, ch)],
                    dbufs[b], dsems[b]).wait()
                pltpu.sync_copy(dbufs[b], accum.at[idxmat.at[j]],
                                add=True)

            start(0, 0)

            def body(jj, carry):
                for b in range(2):
                    j = jj * 2 + b
                    nxt = j + 1

                    @pl.when(nxt < nch)
                    def _():
                        start(nxt, (b + 1) % 2)

                    @pl.when(j < nch)
                    def _():
                        drain(j, b)
                return carry
            lax.fori_loop(0, (nch + 1) // 2, body, 0)
            plsc.subcore_barrier()

            for q in range(rounds):
                rc = sid + q * n_sub

                @pl.when(rc < n_rowch)
                def _dump():
                    r0 = rc * zch
                    pltpu.sync_copy(accum.at[pl.ds(r0, zch)], obuf)
                    pltpu.sync_copy(obuf,
                                    part_hbm.at[cid, p, pl.ds(r0, zch)])
            plsc.subcore_barrier()

    return scatter_k


def _combine_body(s_ref, v2_ref, part_ref, so_ref, vo_ref):
    so_ref[...] = s_ref[...] + part_ref[0, 0] + part_ref[1, 0]
    dv = [part_ref[0, 1 + d] + part_ref[1, 1 + d] for d in range(3)]
    vo_ref[...] = v2_ref[...] + jnp.concatenate(dv, axis=1)


def _combine(s, v2, part, bn):
    n, emb = s.shape
    grid = n // bn
    return pl.pallas_call(
        _combine_body,
        grid=(grid,),
        in_specs=[
            pl.BlockSpec((bn, emb), lambda i: (i, 0)),
            pl.BlockSpec((bn, 3 * emb), lambda i: (i, 0)),
            pl.BlockSpec((2, 4, bn, emb), lambda i: (0, 0, i, 0)),
        ],
        out_specs=[
            pl.BlockSpec((bn, emb), lambda i: (i, 0)),
            pl.BlockSpec((bn, 3 * emb), lambda i: (i, 0)),
        ],
        out_shape=[
            jax.ShapeDtypeStruct((n, emb), jnp.float32),
            jax.ShapeDtypeStruct((n, 3 * emb), jnp.float32),
        ],
    )(s, v2, part)


N_WORKERS = 32
GATHER_CH = 40
SCATTER_CH = 80


def kernel(s, v, edges, r_ij, r_ij_normalized, W1, b1, W2, b2, Wr, br):
    n, emb = s.shape
    E = edges.shape[0]

    edges = edges.astype(jnp.int32)
    src = edges[:, 1]
    dst3 = edges[:, 0].reshape(N_WORKERS, (E // N_WORKERS) // SCATTER_CH,
                               SCATTER_CH)
    v2 = v.reshape(n, 3 * emb)
    r_flat = r_ij.reshape(E)
    rn_t = r_ij_normalized.T

    s_pass = _node_mlp(s, W1, b1, W2, b2, bn=400)
    g, vg = _make_gather(E, 3 * emb, N_WORKERS, GATHER_CH)(s_pass, v2, src)
    d4 = _edge_math(r_flat, rn_t, g, vg, Wr, br, be=512)
    part = _make_scatter(E, n, emb, N_WORKERS, SCATTER_CH)(d4, dst3)
    s_out, v2_out = _combine(s, v2, part, bn=400)
    return (s_out, v2_out.reshape(n, 3, emb))
